# baseline (speedup 1.0000x reference)
import functools

import jax
import jax.numpy as jnp
from jax import lax
from jax.experimental import pallas as pl
from jax.experimental.pallas import tpu as pltpu
from jax.experimental.pallas import tpu_sc as plsc

D_MODEL = 128
CHUNK = 64
K = 4


def _make_sc_gather(n_rows: int, nw: int, nc: int):
    rows_per_w = n_rows // nw
    n_chunks = rows_per_w // CHUNK
    n_pairs = n_chunks // (2 * K)
    mesh = plsc.VectorSubcoreMesh(core_axis_name="c", subcore_axis_name="s")

    @functools.partial(
        pl.kernel,
        mesh=mesh,
        out_type=jax.ShapeDtypeStruct((n_rows, D_MODEL), jnp.float32),
        scratch_types=[
            pltpu.VMEM((n_chunks, CHUNK), jnp.int32),
            pltpu.VMEM((2 * K, CHUNK, D_MODEL), jnp.float32),
            pltpu.VMEM_SHARED((200, D_MODEL), jnp.float32),
            pltpu.SemaphoreType.DMA,
            pltpu.SemaphoreType.DMA,
            pltpu.SemaphoreType.DMA,
            pltpu.SemaphoreType.DMA,
        ],
    )
    def k(idx_hbm, pe_hbm, out_hbm, idx_v, rows_v, pe_sh, gsem_a, gsem_b, ssem_a, ssem_b):
        sid = lax.axis_index("s")
        wid = sid * nc + lax.axis_index("c")
        base = wid * rows_per_w
        @pl.when(sid == 0)
        def _():
            pltpu.sync_copy(pe_hbm, pe_sh)

        pltpu.sync_copy(idx_hbm.at[wid], idx_v)
        plsc.subcore_barrier()

        def start_g(j, b, sem):
            pltpu.async_copy(pe_sh.at[idx_v.at[j]], rows_v.at[b], sem)

        def drain_g(b, sem):
            pltpu.make_async_copy(pe_sh.at[idx_v.at[0]], rows_v.at[b], sem).wait()

        def start_s(j, b, sem):
            pltpu.async_copy(rows_v.at[b], out_hbm.at[pl.ds(base + j * CHUNK, CHUNK)], sem)

        def drain_s(b, sem):
            pltpu.make_async_copy(rows_v.at[b], out_hbm.at[pl.ds(base, CHUNK)], sem).wait()

        for b in range(K):
            start_g(b, b, gsem_a)
        for b in range(K):
            start_g(K + b, K + b, gsem_b)

        def body(p, _):
            c0 = p * 2 * K
            for b in range(K):
                drain_g(b, gsem_a)
            for b in range(K):
                start_s(c0 + b, b, ssem_a)
            for b in range(K):
                drain_g(K + b, gsem_b)
            for b in range(K):
                start_s(c0 + K + b, K + b, ssem_b)
            for b in range(K):
                drain_s(b, ssem_a)
            for b in range(K):
                start_g(c0 + 2 * K + b, b, gsem_a)
            for b in range(K):
                drain_s(K + b, ssem_b)
            for b in range(K):
                start_g(c0 + 3 * K + b, K + b, gsem_b)
            return _

        lax.fori_loop(0, n_pairs - 1, body, None)

        cl = (n_pairs - 1) * 2 * K
        for b in range(K):
            drain_g(b, gsem_a)
        for b in range(K):
            start_s(cl + b, b, ssem_a)
        for b in range(K):
            drain_g(K + b, gsem_b)
        for b in range(K):
            start_s(cl + K + b, K + b, ssem_b)
        for b in range(K):
            drain_s(b, ssem_a)
        for b in range(K):
            drain_s(K + b, ssem_b)

    return k


TC_BLOCK = 1024
K_PAD = 256


def _tc_body(xr, hir, lor, outr):
    idx = xr[0]
    kiota = lax.broadcasted_iota(jnp.int32, (K_PAD, 1), 0)
    for j in range(8):
        row = idx[j:j + 1, :]
        oht = (kiota == row).astype(jnp.bfloat16)
        acc = lax.dot_general(oht, hir[...], (((0,), (0,)), ((), ())),
                              preferred_element_type=jnp.float32)
        acc += lax.dot_general(oht, lor[...], (((0,), (0,)), ((), ())),
                               preferred_element_type=jnp.float32)
        outr[pl.ds(j * 128, 128), :] = acc


def _tc_onehot(x_rows, pe):
    n = x_rows.shape[0]
    g = n // TC_BLOCK
    pe_pad = jnp.pad(pe, ((0, K_PAD - pe.shape[0]), (0, 0)))
    hi = pe_pad.astype(jnp.bfloat16)
    lo = (pe_pad - hi.astype(jnp.float32)).astype(jnp.bfloat16)
    xg = x_rows.reshape(g, 8, 128).astype(jnp.int32)
    return pl.pallas_call(
        _tc_body,
        grid=(g,),
        in_specs=[
            pl.BlockSpec((1, 8, 128), lambda i: (i, 0, 0)),
            pl.BlockSpec((K_PAD, D_MODEL), lambda i: (0, 0)),
            pl.BlockSpec((K_PAD, D_MODEL), lambda i: (0, 0)),
        ],
        out_specs=pl.BlockSpec((TC_BLOCK, D_MODEL), lambda i: (i, 0)),
        out_shape=jax.ShapeDtypeStruct((n, D_MODEL), jnp.float32),
    )(xg, hi, lo)


def kernel(x, pe):
    b, s = x.shape
    n_rows = b * s
    out = _tc_onehot(x.reshape(n_rows), pe)
    return out.reshape(b, s, D_MODEL)


# speedup vs baseline: 1.1243x; 1.1243x over previous
import functools

import jax
import jax.numpy as jnp
from jax import lax
from jax.experimental import pallas as pl
from jax.experimental.pallas import tpu as pltpu
from jax.experimental.pallas import tpu_sc as plsc

D_MODEL = 128
CHUNK = 64
K = 4


def _make_sc_gather(n_rows: int, nw: int, nc: int):
    rows_per_w = n_rows // nw
    n_chunks = rows_per_w // CHUNK
    n_pairs = n_chunks // (2 * K)
    mesh = plsc.VectorSubcoreMesh(core_axis_name="c", subcore_axis_name="s")

    @functools.partial(
        pl.kernel,
        mesh=mesh,
        out_type=jax.ShapeDtypeStruct((n_rows, D_MODEL), jnp.float32),
        scratch_types=[
            pltpu.VMEM((n_chunks, CHUNK), jnp.int32),
            pltpu.VMEM((2 * K, CHUNK, D_MODEL), jnp.float32),
            pltpu.VMEM_SHARED((200, D_MODEL), jnp.float32),
            pltpu.SemaphoreType.DMA,
            pltpu.SemaphoreType.DMA,
            pltpu.SemaphoreType.DMA,
            pltpu.SemaphoreType.DMA,
        ],
    )
    def k(idx_hbm, pe_hbm, out_hbm, idx_v, rows_v, pe_sh, gsem_a, gsem_b, ssem_a, ssem_b):
        sid = lax.axis_index("s")
        wid = sid * nc + lax.axis_index("c")
        base = wid * rows_per_w
        @pl.when(sid == 0)
        def _():
            pltpu.sync_copy(pe_hbm, pe_sh)

        pltpu.sync_copy(idx_hbm.at[wid], idx_v)
        plsc.subcore_barrier()

        def start_g(j, b, sem):
            pltpu.async_copy(pe_sh.at[idx_v.at[j]], rows_v.at[b], sem)

        def drain_g(b, sem):
            pltpu.make_async_copy(pe_sh.at[idx_v.at[0]], rows_v.at[b], sem).wait()

        def start_s(j, b, sem):
            pltpu.async_copy(rows_v.at[b], out_hbm.at[pl.ds(base + j * CHUNK, CHUNK)], sem)

        def drain_s(b, sem):
            pltpu.make_async_copy(rows_v.at[b], out_hbm.at[pl.ds(base, CHUNK)], sem).wait()

        for b in range(K):
            start_g(b, b, gsem_a)
        for b in range(K):
            start_g(K + b, K + b, gsem_b)

        def body(p, _):
            c0 = p * 2 * K
            for b in range(K):
                drain_g(b, gsem_a)
            for b in range(K):
                start_s(c0 + b, b, ssem_a)
            for b in range(K):
                drain_g(K + b, gsem_b)
            for b in range(K):
                start_s(c0 + K + b, K + b, ssem_b)
            for b in range(K):
                drain_s(b, ssem_a)
            for b in range(K):
                start_g(c0 + 2 * K + b, b, gsem_a)
            for b in range(K):
                drain_s(K + b, ssem_b)
            for b in range(K):
                start_g(c0 + 3 * K + b, K + b, gsem_b)
            return _

        lax.fori_loop(0, n_pairs - 1, body, None)

        cl = (n_pairs - 1) * 2 * K
        for b in range(K):
            drain_g(b, gsem_a)
        for b in range(K):
            start_s(cl + b, b, ssem_a)
        for b in range(K):
            drain_g(K + b, gsem_b)
        for b in range(K):
            start_s(cl + K + b, K + b, ssem_b)
        for b in range(K):
            drain_s(b, ssem_a)
        for b in range(K):
            drain_s(K + b, ssem_b)

    return k


TC_BLOCK = 1024
K_PAD = 256


def _tc_body(xr, hir, lor, outr):
    idx = xr[0]
    kiota = lax.broadcasted_iota(jnp.int32, (K_PAD, 1), 0)
    for j in range(8):
        row = idx[j:j + 1, :]
        oht = (kiota == row).astype(jnp.bfloat16)
        acc = lax.dot_general(oht, hir[...], (((0,), (0,)), ((), ())),
                              preferred_element_type=jnp.float32)
        acc += lax.dot_general(oht, lor[...], (((0,), (0,)), ((), ())),
                               preferred_element_type=jnp.float32)
        outr[pl.ds(j * 128, 128), :] = acc


def _tc_onehot(x_rows, pe):
    n = x_rows.shape[0]
    g = n // TC_BLOCK
    pe_pad = jnp.pad(pe, ((0, K_PAD - pe.shape[0]), (0, 0)))
    bits = lax.bitcast_convert_type(pe_pad, jnp.uint32)
    hb = (bits + jnp.uint32(0x8000)) & jnp.uint32(0xFFFF0000)
    hi_f = lax.bitcast_convert_type(hb, jnp.float32)
    hi = hi_f.astype(jnp.bfloat16)
    lo = (pe_pad - hi_f).astype(jnp.bfloat16)
    xg = x_rows.reshape(g, 8, 128).astype(jnp.int32)
    return pl.pallas_call(
        _tc_body,
        grid=(g,),
        in_specs=[
            pl.BlockSpec((1, 8, 128), lambda i: (i, 0, 0)),
            pl.BlockSpec((K_PAD, D_MODEL), lambda i: (0, 0)),
            pl.BlockSpec((K_PAD, D_MODEL), lambda i: (0, 0)),
        ],
        out_specs=pl.BlockSpec((TC_BLOCK, D_MODEL), lambda i: (i, 0)),
        out_shape=jax.ShapeDtypeStruct((n, D_MODEL), jnp.float32),
    )(xg, hi, lo)


def kernel(x, pe):
    b, s = x.shape
    n_rows = b * s
    info = plsc.get_sparse_core_info()
    nc, ns = info.num_cores, info.num_subcores
    nw = nc * ns
    r_sc = 557056
    xf = x.reshape(n_rows)
    idx = xf[:r_sc].reshape(nw, r_sc // nw // CHUNK, CHUNK).astype(jnp.int32)
    out_sc = _make_sc_gather(r_sc, nw, nc)(idx, pe)
    out_tc = _tc_onehot(xf[r_sc:], pe)
    out = jnp.concatenate([out_sc, out_tc], axis=0)
    return out.reshape(b, s, D_MODEL)


# speedup vs baseline: 2.1077x; 1.8747x over previous
import functools

import jax
import jax.numpy as jnp
from jax import lax
from jax.experimental import pallas as pl
from jax.experimental.pallas import tpu as pltpu
from jax.experimental.pallas import tpu_sc as plsc

D_MODEL = 128
CHUNK = 128
K = 2


def _make_sc_gather(n_rows: int, nw: int, nc: int):
    rows_per_w = n_rows // nw
    n_chunks = rows_per_w // CHUNK
    n_pairs = n_chunks // (2 * K)
    mesh = plsc.VectorSubcoreMesh(core_axis_name="c", subcore_axis_name="s")

    @functools.partial(
        pl.kernel,
        mesh=mesh,
        out_type=jax.ShapeDtypeStruct((n_rows, D_MODEL), jnp.float32),
        scratch_types=[
            pltpu.VMEM((n_chunks, CHUNK), jnp.int32),
            pltpu.VMEM((2 * K, CHUNK, D_MODEL), jnp.float32),
            pltpu.VMEM_SHARED((200, D_MODEL), jnp.float32),
            pltpu.SemaphoreType.DMA,
            pltpu.SemaphoreType.DMA,
            pltpu.SemaphoreType.DMA,
            pltpu.SemaphoreType.DMA,
        ],
    )
    def k(idx_hbm, pe_hbm, out_hbm, idx_v, rows_v, pe_sh, gsem_a, gsem_b, ssem_a, ssem_b):
        sid = lax.axis_index("s")
        wid = sid * nc + lax.axis_index("c")
        base = wid * rows_per_w
        @pl.when(sid == 0)
        def _():
            pltpu.sync_copy(pe_hbm, pe_sh)

        pltpu.sync_copy(idx_hbm.at[wid], idx_v)
        plsc.subcore_barrier()

        def start_g(j, b, sem):
            pltpu.async_copy(pe_sh.at[idx_v.at[j]], rows_v.at[b], sem)

        def drain_g(b, sem):
            pltpu.make_async_copy(pe_sh.at[idx_v.at[0]], rows_v.at[b], sem).wait()

        def start_s(j, b, sem):
            pltpu.async_copy(rows_v.at[b], out_hbm.at[pl.ds(base + j * CHUNK, CHUNK)], sem)

        def drain_s(b, sem):
            pltpu.make_async_copy(rows_v.at[b], out_hbm.at[pl.ds(base, CHUNK)], sem).wait()

        for b in range(K):
            start_g(b, b, gsem_a)
        for b in range(K):
            start_g(K + b, K + b, gsem_b)

        def body(p, _):
            c0 = p * 2 * K
            for b in range(K):
                drain_g(b, gsem_a)
            for b in range(K):
                start_s(c0 + b, b, ssem_a)
            for b in range(K):
                drain_g(K + b, gsem_b)
            for b in range(K):
                start_s(c0 + K + b, K + b, ssem_b)
            for b in range(K):
                drain_s(b, ssem_a)
            for b in range(K):
                start_g(c0 + 2 * K + b, b, gsem_a)
            for b in range(K):
                drain_s(K + b, ssem_b)
            for b in range(K):
                start_g(c0 + 3 * K + b, K + b, gsem_b)
            return _

        lax.fori_loop(0, n_pairs - 1, body, None)

        cl = (n_pairs - 1) * 2 * K
        for b in range(K):
            drain_g(b, gsem_a)
        for b in range(K):
            start_s(cl + b, b, ssem_a)
        for b in range(K):
            drain_g(K + b, gsem_b)
        for b in range(K):
            start_s(cl + K + b, K + b, ssem_b)
        for b in range(K):
            drain_s(b, ssem_a)
        for b in range(K):
            drain_s(K + b, ssem_b)

    return k


TC_BLOCK = 1024
K_PAD = 256


def _tc_body(xr, hir, lor, outr):
    idx = xr[0]
    kiota = lax.broadcasted_iota(jnp.int32, (K_PAD, 1), 0)
    for j in range(8):
        row = idx[j:j + 1, :]
        oht = (kiota == row).astype(jnp.bfloat16)
        acc = lax.dot_general(oht, hir[...], (((0,), (0,)), ((), ())),
                              preferred_element_type=jnp.float32)
        acc += lax.dot_general(oht, lor[...], (((0,), (0,)), ((), ())),
                               preferred_element_type=jnp.float32)
        outr[pl.ds(j * 128, 128), :] = acc


def _tc_onehot(x_rows, pe):
    n = x_rows.shape[0]
    g = n // TC_BLOCK
    pe_pad = jnp.pad(pe, ((0, K_PAD - pe.shape[0]), (0, 0)))
    bits = lax.bitcast_convert_type(pe_pad, jnp.uint32)
    hb = (bits + jnp.uint32(0x8000)) & jnp.uint32(0xFFFF0000)
    hi_f = lax.bitcast_convert_type(hb, jnp.float32)
    hi = hi_f.astype(jnp.bfloat16)
    lo = (pe_pad - hi_f).astype(jnp.bfloat16)
    xg = x_rows.reshape(g, 8, 128).astype(jnp.int32)
    return pl.pallas_call(
        _tc_body,
        grid=(g,),
        in_specs=[
            pl.BlockSpec((1, 8, 128), lambda i: (i, 0, 0)),
            pl.BlockSpec((K_PAD, D_MODEL), lambda i: (0, 0)),
            pl.BlockSpec((K_PAD, D_MODEL), lambda i: (0, 0)),
        ],
        out_specs=pl.BlockSpec((TC_BLOCK, D_MODEL), lambda i: (i, 0)),
        out_shape=jax.ShapeDtypeStruct((n, D_MODEL), jnp.float32),
    )(xg, hi, lo)


def kernel(x, pe):
    b, s = x.shape
    n_rows = b * s
    info = plsc.get_sparse_core_info()
    nc, ns = info.num_cores, info.num_subcores
    nw = nc * ns
    rows_per_w = n_rows // nw
    idx = x.reshape(nw, rows_per_w // CHUNK, CHUNK).astype(jnp.int32)
    out = _make_sc_gather(n_rows, nw, nc)(idx, pe)
    return out.reshape(b, s, D_MODEL)
